# baseline (speedup 1.0000x reference)
import jax
import jax.numpy as jnp
from jax.experimental import pallas as pl
from jax.experimental.pallas import tpu as pltpu

_NC = 4
_BT = 16384 // _NC


def _embed_kernel(x_hbm, w_ref, b_ref, o_hbm, x_vmem, o_vmem, in_sems, out_sem):
    in_copies = []
    for i in range(_NC):
        c = pltpu.make_async_copy(
            x_hbm.at[pl.ds(i * _BT, _BT), :],
            x_vmem.at[pl.ds(i * _BT, _BT), :],
            in_sems.at[i],
        )
        c.start()
        in_copies.append(c)
    for i in range(_NC):
        in_copies[i].wait()
        o_vmem[pl.ds(i * _BT, _BT), :] = (
            jnp.dot(
                x_vmem[pl.ds(i * _BT, _BT), :],
                w_ref[:],
                preferred_element_type=jnp.float32,
            )
            + b_ref[:]
        )
    wc = pltpu.make_async_copy(o_vmem, o_hbm, out_sem)
    wc.start()
    wc.wait()


@jax.jit
def kernel(x, W0, b0, W1, b1, W2, b2, W3, b3):
    G, H = W0.shape
    n = 4
    D = G * n
    O = H * n
    Wb = jnp.zeros((D, O), x.dtype)
    for i, W in enumerate((W0, W1, W2, W3)):
        Wb = jax.lax.dynamic_update_slice(Wb, W, (i * G, i * H))
    bb = jnp.concatenate([b0, b1, b2, b3]).reshape(1, O)

    B = x.shape[0]
    return pl.pallas_call(
        _embed_kernel,
        in_specs=[
            pl.BlockSpec(memory_space=pltpu.MemorySpace.HBM),
            pl.BlockSpec(memory_space=pltpu.VMEM),
            pl.BlockSpec(memory_space=pltpu.VMEM),
        ],
        out_specs=pl.BlockSpec(memory_space=pltpu.MemorySpace.HBM),
        out_shape=jax.ShapeDtypeStruct((B, O), x.dtype),
        scratch_shapes=[
            pltpu.VMEM((B, D), x.dtype),
            pltpu.VMEM((B, O), x.dtype),
            pltpu.SemaphoreType.DMA((_NC,)),
            pltpu.SemaphoreType.DMA,
        ],
    )(x, Wb, bb)


# speedup vs baseline: 1.0022x; 1.0022x over previous
import jax
import jax.numpy as jnp
from jax.experimental import pallas as pl
from jax.experimental.pallas import tpu as pltpu

_NC = 4
_BT = 16384 // _NC


def _embed_kernel(x_hbm, w_ref, b_ref, o_hbm, x_vmem, o_vmem, in_sems, out_sems):
    in_copies = []
    for i in range(_NC):
        c = pltpu.make_async_copy(
            x_hbm.at[pl.ds(i * _BT, _BT), :],
            x_vmem.at[pl.ds(i * _BT, _BT), :],
            in_sems.at[i],
        )
        c.start()
        in_copies.append(c)
    for c in in_copies:
        c.wait()
    out_copies = []
    for i in range(_NC):
        o_vmem[pl.ds(i * _BT, _BT), :] = (
            jnp.dot(
                x_vmem[pl.ds(i * _BT, _BT), :],
                w_ref[:],
                preferred_element_type=jnp.float32,
            )
            + b_ref[:]
        )
        c = pltpu.make_async_copy(
            o_vmem.at[pl.ds(i * _BT, _BT), :],
            o_hbm.at[pl.ds(i * _BT, _BT), :],
            out_sems.at[i],
        )
        c.start()
        out_copies.append(c)
    for c in out_copies:
        c.wait()


@jax.jit
def kernel(x, W0, b0, W1, b1, W2, b2, W3, b3):
    G, H = W0.shape
    n = 4
    D = G * n
    O = H * n
    Wb = jnp.zeros((D, O), x.dtype)
    for i, W in enumerate((W0, W1, W2, W3)):
        Wb = jax.lax.dynamic_update_slice(Wb, W, (i * G, i * H))
    bb = jnp.concatenate([b0, b1, b2, b3]).reshape(1, O)

    B = x.shape[0]
    return pl.pallas_call(
        _embed_kernel,
        in_specs=[
            pl.BlockSpec(memory_space=pltpu.MemorySpace.HBM),
            pl.BlockSpec(memory_space=pltpu.VMEM),
            pl.BlockSpec(memory_space=pltpu.VMEM),
        ],
        out_specs=pl.BlockSpec(memory_space=pltpu.MemorySpace.HBM),
        out_shape=jax.ShapeDtypeStruct((B, O), x.dtype),
        scratch_shapes=[
            pltpu.VMEM((B, D), x.dtype),
            pltpu.VMEM((B, O), x.dtype),
            pltpu.SemaphoreType.DMA((_NC,)),
            pltpu.SemaphoreType.DMA((_NC,)),
        ],
    )(x, Wb, bb)


# speedup vs baseline: 1.6630x; 1.6594x over previous
import jax
import jax.numpy as jnp
from jax.experimental import pallas as pl

_BT = 8192


def _embed_kernel(x_ref, w0, b0, w1, b1, w2, b2, w3, b3, o_ref):
    ws = [w0, w1, w2, w3]
    bs = [b0, b1, b2, b3]
    wb = jnp.concatenate(
        [
            jnp.pad(ws[i][...], ((0, 0), (32 * i, 96 - 32 * i)))
            for i in range(4)
        ],
        axis=0,
    )
    bb = jnp.concatenate([b[...] for b in bs], axis=1)
    o_ref[...] = (
        jnp.dot(x_ref[...], wb, preferred_element_type=jnp.float32) + bb
    )


@jax.jit
def kernel(x, W0, b0, W1, b1, W2, b2, W3, b3):
    G, H = W0.shape
    D = G * 4
    O = H * 4
    B = x.shape[0]
    wspec = pl.BlockSpec((G, H), lambda i: (0, 0))
    bspec = pl.BlockSpec((1, H), lambda i: (0, 0))
    return pl.pallas_call(
        _embed_kernel,
        grid=(B // _BT,),
        in_specs=[
            pl.BlockSpec((_BT, D), lambda i: (i, 0)),
            wspec, bspec, wspec, bspec, wspec, bspec, wspec, bspec,
        ],
        out_specs=pl.BlockSpec((_BT, O), lambda i: (i, 0)),
        out_shape=jax.ShapeDtypeStruct((B, O), x.dtype),
    )(
        x,
        W0, b0.reshape(1, H),
        W1, b1.reshape(1, H),
        W2, b2.reshape(1, H),
        W3, b3.reshape(1, H),
    )
